# Initial kernel scaffold; baseline (speedup 1.0000x reference)
import jax
import jax.numpy as jnp
from jax.experimental import pallas as pl


def kernel(bboxes, P0, P1, P2, P3):
    raise NotImplementedError("write your pallas kernel here")


# speedup vs baseline: 65.6235x; 65.6235x over previous
import functools
import jax
import jax.numpy as jnp
from jax import lax
from jax.experimental import pallas as pl
from jax.experimental.pallas import tpu as pltpu
from jax.experimental.pallas import tpu_sc as plsc

_POOL = 7
_MAX_TOK = 784.0
_MIN_TOK = 196.0

_HWS = ((100, 152), (50, 76), (25, 38), (13, 19))
_BASES = (0, 15200, 19000, 19952)
_TABLE_ROWS = 20224

_NB = 1024
_PER_WORKER = 32
_K = 16
_NPIX = _POOL * _POOL
_NJ = _NPIX * _K
_CHUNK = 112
_NCHUNK = _NJ // _CHUNK


def _relayout_body(p0, p1, p2, p3, out):
    out[pl.ds(0, 15200), :] = p0[...].T
    out[pl.ds(15200, 3800), :] = p1[...].T
    out[pl.ds(19000, 952), :] = p2[...].T
    out[pl.ds(19952, 248), :] = p3[...].T


def _build_table(P0, P1, P2, P3):
    C = P0.shape[0]
    f0 = P0.reshape(C, -1)
    f1 = P1.reshape(C, -1)
    f2 = jnp.pad(P2.reshape(C, -1), ((0, 0), (0, 2)))
    f3 = jnp.pad(P3.reshape(C, -1), ((0, 0), (0, 1)))
    return pl.pallas_call(
        _relayout_body,
        out_shape=jax.ShapeDtypeStruct((_TABLE_ROWS, C), jnp.float32),
    )(f0, f1, f2, f3)


def _corner(v, cidx, ext):
    v = jnp.maximum(v, 0.0)
    vl = jnp.floor(v)
    edge = vl >= (ext - 1.0)
    lo = jnp.where(edge, ext - 1.0, vl)
    hi = jnp.minimum(lo + 1.0, ext - 1.0)
    frac = jnp.where(edge, 0.0, v - vl)
    wgt = jnp.where(cidx == 1, frac, 1.0 - frac)
    pos = jnp.where(cidx == 1, hi, lo)
    return pos, wgt


def _addr_body(bb_ref, idx_ref, w_ref):
    bb = bb_ref[...]
    bx1 = bb[:, 0:1]
    by1 = bb[:, 1:2]
    bx2 = bb[:, 2:3]
    by2 = bb[:, 3:4]
    area = (bx2 - bx1) * (by2 - by1)

    choice = jnp.full_like(area, 3.0)
    for lvl in (2, 1, 0):
        h, w = _HWS[lvl]
        tok = area * float(h * w)
        m = (tok < _MAX_TOK) & (tok >= _MIN_TOK)
        choice = jnp.where(m, float(lvl), choice)

    def sel(vals):
        r = jnp.full_like(area, vals[3])
        for lvl in (2, 1, 0):
            r = jnp.where(choice == float(lvl), vals[lvl], r)
        return r

    Hc = sel([float(h) for h, _ in _HWS])
    Wc = sel([float(w) for _, w in _HWS])
    basec = sel([float(b) for b in _BASES])

    x1 = bx1 * Wc - 0.5
    y1 = by1 * Hc - 0.5
    bin_h = (by2 * Hc - 0.5 - y1) / float(_POOL)
    bin_w = (bx2 * Wc - 0.5 - x1) / float(_POOL)

    B = bb.shape[0]
    j = lax.broadcasted_iota(jnp.int32, (B, _NJ), 1)
    ph = (j // 112).astype(jnp.float32)
    pw = ((j // 16) % 7).astype(jnp.float32)
    sy = ((j // 8) % 2).astype(jnp.float32)
    cy = (j // 4) % 2
    sx = ((j // 2) % 2).astype(jnp.float32)
    cx = j % 2

    yf = y1 + (ph + (sy + 0.5) * 0.5) * bin_h
    xf = x1 + (pw + (sx + 0.5) * 0.5) * bin_w
    ypos, wy = _corner(yf, cy, Hc)
    xpos, wx = _corner(xf, cx, Wc)

    idx_ref[...] = (basec + ypos * Wc + xpos).astype(jnp.int32)
    w_ref[...] = wy * wx * 0.25


def _gen_addresses(bboxes_padded):
    grid = 8
    blk = _NB // grid
    return pl.pallas_call(
        _addr_body,
        grid=(grid,),
        in_specs=[pl.BlockSpec((blk, 4), lambda i: (i, 0))],
        out_specs=[
            pl.BlockSpec((blk, _NJ), lambda i: (i, 0)),
            pl.BlockSpec((blk, _NJ), lambda i: (i, 0)),
        ],
        out_shape=[
            jax.ShapeDtypeStruct((_NB, _NJ), jnp.int32),
            jax.ShapeDtypeStruct((_NB, _NJ), jnp.float32),
        ],
    )(bboxes_padded)


def _sc_body(table_hbm, idx_hbm, w_hbm, out_hbm, idx_v, w_v, rows_v, out_v, sem):
    wid = lax.axis_index("s") * 2 + lax.axis_index("c")
    base = wid * _PER_WORKER

    def box_body(i, carry):
        b = base + i
        pltpu.sync_copy(idx_hbm.at[b], idx_v)
        pltpu.sync_copy(w_hbm.at[b], w_v)
        copies = []
        for c in range(_NCHUNK):
            copies.append(
                pltpu.async_copy(
                    table_hbm.at[idx_v.at[c]],
                    rows_v.at[pl.ds(c * _CHUNK, _CHUNK)],
                    sem,
                )
            )
        for cp in copies:
            cp.wait()

        def pix_body(p, carry2):
            r0 = p * _K
            wvec = w_v[pl.ds(r0, _K)]
            accs = [jnp.zeros((16,), jnp.float32) for _ in range(4)]
            for k in range(_K):
                ws = wvec[k]
                for q in range(4):
                    accs[q] = accs[q] + ws * rows_v[r0 + k, pl.ds(q * 16, 16)]
            for q in range(4):
                out_v[p, pl.ds(q * 16, 16)] = accs[q]
            return carry2

        lax.fori_loop(0, _NPIX, pix_body, 0)
        pltpu.sync_copy(out_v, out_hbm.at[b])
        return carry

    lax.fori_loop(0, _PER_WORKER, box_body, 0)


def _sc_gather(table, idx, w):
    mesh = plsc.VectorSubcoreMesh(core_axis_name="c", subcore_axis_name="s")
    kern = functools.partial(
        pl.kernel,
        mesh=mesh,
        compiler_params=pltpu.CompilerParams(use_tc_tiling_on_sc=False),
        out_type=jax.ShapeDtypeStruct((_NB, _NPIX, 64), jnp.float32),
        scratch_types=[
            pltpu.VMEM((_NCHUNK, _CHUNK), jnp.int32),
            pltpu.VMEM((_NJ,), jnp.float32),
            pltpu.VMEM((_NJ, 64), jnp.float32),
            pltpu.VMEM((_NPIX, 64), jnp.float32),
            pltpu.SemaphoreType.DMA,
        ],
    )(_sc_body)
    return kern(table, idx.reshape(_NB, _NCHUNK, _CHUNK), w)


def _xpose_body(i_ref, o_ref):
    o_ref[...] = jnp.swapaxes(i_ref[...], 1, 2)


def _xpose(out_pc):
    grid = 16
    blk = _NB // grid
    return pl.pallas_call(
        _xpose_body,
        grid=(grid,),
        in_specs=[pl.BlockSpec((blk, _NPIX, 64), lambda i: (i, 0, 0))],
        out_specs=pl.BlockSpec((blk, 64, _NPIX), lambda i: (i, 0, 0)),
        out_shape=jax.ShapeDtypeStruct((_NB, 64, _NPIX), jnp.float32),
    )(out_pc)


def kernel(bboxes, P0, P1, P2, P3):
    N = bboxes.shape[0]
    table = _build_table(P0, P1, P2, P3)
    bb = jnp.pad(bboxes, ((0, _NB - N), (0, 0)))
    idx, w = _gen_addresses(bb)
    out = _sc_gather(table, idx, w)
    return _xpose(out).reshape(_NB, 64, _POOL, _POOL)[:N]


# speedup vs baseline: 76.6422x; 1.1679x over previous
import functools
import jax
import jax.numpy as jnp
from jax import lax
from jax.experimental import pallas as pl
from jax.experimental.pallas import tpu as pltpu
from jax.experimental.pallas import tpu_sc as plsc

_POOL = 7
_MAX_TOK = 784.0
_MIN_TOK = 196.0

_HWS = ((100, 152), (50, 76), (25, 38), (13, 19))
_BASES = (0, 15200, 19000, 19952)
_TABLE_ROWS = 20224

_NB = 1024
_PER_WORKER = 32
_K = 16
_NPIX = _POOL * _POOL
_NJ = _NPIX * _K
_CHUNK = 112
_NCHUNK = _NJ // _CHUNK


def _relayout_body(p0, p1, p2, p3, out):
    out[pl.ds(0, 15200), :] = p0[...].T
    out[pl.ds(15200, 3800), :] = p1[...].T
    out[pl.ds(19000, 952), :] = p2[...].T
    out[pl.ds(19952, 248), :] = p3[...].T


def _build_table(P0, P1, P2, P3):
    C = P0.shape[0]
    f0 = P0.reshape(C, -1)
    f1 = P1.reshape(C, -1)
    f2 = jnp.pad(P2.reshape(C, -1), ((0, 0), (0, 2)))
    f3 = jnp.pad(P3.reshape(C, -1), ((0, 0), (0, 1)))
    return pl.pallas_call(
        _relayout_body,
        out_shape=jax.ShapeDtypeStruct((_TABLE_ROWS, C), jnp.float32),
    )(f0, f1, f2, f3)


def _corner(v, cidx, ext):
    v = jnp.maximum(v, 0.0)
    vl = jnp.floor(v)
    edge = vl >= (ext - 1.0)
    lo = jnp.where(edge, ext - 1.0, vl)
    hi = jnp.minimum(lo + 1.0, ext - 1.0)
    frac = jnp.where(edge, 0.0, v - vl)
    wgt = jnp.where(cidx == 1, frac, 1.0 - frac)
    pos = jnp.where(cidx == 1, hi, lo)
    return pos, wgt


def _addr_body(bb_ref, idx_ref, w_ref):
    bb = bb_ref[...]
    bx1 = bb[:, 0:1]
    by1 = bb[:, 1:2]
    bx2 = bb[:, 2:3]
    by2 = bb[:, 3:4]
    area = (bx2 - bx1) * (by2 - by1)

    choice = jnp.full_like(area, 3.0)
    for lvl in (2, 1, 0):
        h, w = _HWS[lvl]
        tok = area * float(h * w)
        m = (tok < _MAX_TOK) & (tok >= _MIN_TOK)
        choice = jnp.where(m, float(lvl), choice)

    def sel(vals):
        r = jnp.full_like(area, vals[3])
        for lvl in (2, 1, 0):
            r = jnp.where(choice == float(lvl), vals[lvl], r)
        return r

    Hc = sel([float(h) for h, _ in _HWS])
    Wc = sel([float(w) for _, w in _HWS])
    basec = sel([float(b) for b in _BASES])

    x1 = bx1 * Wc - 0.5
    y1 = by1 * Hc - 0.5
    bin_h = (by2 * Hc - 0.5 - y1) / float(_POOL)
    bin_w = (bx2 * Wc - 0.5 - x1) / float(_POOL)

    B = bb.shape[0]
    j = lax.broadcasted_iota(jnp.int32, (B, _NJ), 1)
    ph = (j // 112).astype(jnp.float32)
    pw = ((j // 16) % 7).astype(jnp.float32)
    sy = ((j // 8) % 2).astype(jnp.float32)
    cy = (j // 4) % 2
    sx = ((j // 2) % 2).astype(jnp.float32)
    cx = j % 2

    yf = y1 + (ph + (sy + 0.5) * 0.5) * bin_h
    xf = x1 + (pw + (sx + 0.5) * 0.5) * bin_w
    ypos, wy = _corner(yf, cy, Hc)
    xpos, wx = _corner(xf, cx, Wc)

    idx_ref[...] = (basec + ypos * Wc + xpos).astype(jnp.int32)
    w_ref[...] = wy * wx * 0.25


def _gen_addresses(bboxes_padded):
    grid = 8
    blk = _NB // grid
    return pl.pallas_call(
        _addr_body,
        grid=(grid,),
        in_specs=[pl.BlockSpec((blk, 4), lambda i: (i, 0))],
        out_specs=[
            pl.BlockSpec((blk, _NJ), lambda i: (i, 0)),
            pl.BlockSpec((blk, _NJ), lambda i: (i, 0)),
        ],
        out_shape=[
            jax.ShapeDtypeStruct((_NB, _NJ), jnp.int32),
            jax.ShapeDtypeStruct((_NB, _NJ), jnp.float32),
        ],
    )(bboxes_padded)


def _sc_body(table_hbm, idx_hbm, w_hbm, out_hbm, idx_v, w_v, rows_v, out_v,
             gsem0, gsem1, osem0, osem1):
    wid = lax.axis_index("s") * 2 + lax.axis_index("c")
    base = wid * _PER_WORKER
    gsems = (gsem0, gsem1)
    osems = (osem0, osem1)

    def fire(b, buf):
        pltpu.sync_copy(idx_hbm.at[b], idx_v.at[buf])
        pltpu.sync_copy(w_hbm.at[b], w_v.at[buf])
        for c in range(_NCHUNK):
            pltpu.async_copy(
                table_hbm.at[idx_v.at[buf, c]],
                rows_v.at[buf, pl.ds(c * _CHUNK, _CHUNK)],
                gsems[buf],
            )

    def drain_gather(buf):
        for c in range(_NCHUNK):
            pltpu.make_async_copy(
                table_hbm.at[idx_v.at[buf, c]],
                rows_v.at[buf, pl.ds(c * _CHUNK, _CHUNK)],
                gsems[buf],
            ).wait()

    def drain_out(b, buf):
        pltpu.make_async_copy(out_v.at[buf], out_hbm.at[b], osems[buf]).wait()

    def compute(b, buf):
        def pix_body(p, carry2):
            r0 = p * _K
            wvec = w_v[buf, pl.ds(r0, _K)]
            accs = [jnp.zeros((16,), jnp.float32) for _ in range(4)]
            for k in range(_K):
                ws = wvec[k]
                for q in range(4):
                    accs[q] = accs[q] + ws * rows_v[buf, r0 + k, pl.ds(q * 16, 16)]
            for q in range(4):
                out_v[buf, p, pl.ds(q * 16, 16)] = accs[q]
            return carry2

        lax.fori_loop(0, _NPIX, pix_body, 0)
        pltpu.async_copy(out_v.at[buf], out_hbm.at[b], osems[buf])

    npair = _PER_WORKER // 2
    fire(base, 0)

    def pair_body(g, carry):
        b0 = base + 2 * g
        fire(b0 + 1, 1)
        drain_gather(0)

        @pl.when(g > 0)
        def _():
            drain_out(b0, 0)

        compute(b0, 0)

        @pl.when(g < npair - 1)
        def _():
            fire(b0 + 2, 0)

        drain_gather(1)

        @pl.when(g > 0)
        def _():
            drain_out(b0 + 1, 1)

        compute(b0 + 1, 1)
        return carry

    lax.fori_loop(0, npair, pair_body, 0)
    drain_out(base, 0)
    drain_out(base, 1)


def _sc_gather(table, idx, w):
    mesh = plsc.VectorSubcoreMesh(core_axis_name="c", subcore_axis_name="s")
    kern = functools.partial(
        pl.kernel,
        mesh=mesh,
        compiler_params=pltpu.CompilerParams(use_tc_tiling_on_sc=False),
        out_type=jax.ShapeDtypeStruct((_NB, _NPIX, 64), jnp.float32),
        scratch_types=[
            pltpu.VMEM((2, _NCHUNK, _CHUNK), jnp.int32),
            pltpu.VMEM((2, _NJ), jnp.float32),
            pltpu.VMEM((2, _NJ, 64), jnp.float32),
            pltpu.VMEM((2, _NPIX, 64), jnp.float32),
            pltpu.SemaphoreType.DMA,
            pltpu.SemaphoreType.DMA,
            pltpu.SemaphoreType.DMA,
            pltpu.SemaphoreType.DMA,
        ],
    )(_sc_body)
    return kern(table, idx.reshape(_NB, _NCHUNK, _CHUNK), w)


def _xpose_body(i_ref, o_ref):
    o_ref[...] = jnp.swapaxes(i_ref[...], 1, 2)


def _xpose(out_pc):
    grid = 16
    blk = _NB // grid
    return pl.pallas_call(
        _xpose_body,
        grid=(grid,),
        in_specs=[pl.BlockSpec((blk, _NPIX, 64), lambda i: (i, 0, 0))],
        out_specs=pl.BlockSpec((blk, 64, _NPIX), lambda i: (i, 0, 0)),
        out_shape=jax.ShapeDtypeStruct((_NB, 64, _NPIX), jnp.float32),
    )(out_pc)


def kernel(bboxes, P0, P1, P2, P3):
    N = bboxes.shape[0]
    table = _build_table(P0, P1, P2, P3)
    bb = jnp.pad(bboxes, ((0, _NB - N), (0, 0)))
    idx, w = _gen_addresses(bb)
    out = _sc_gather(table, idx, w)
    return _xpose(out).reshape(_NB, 64, _POOL, _POOL)[:N]


# speedup vs baseline: 84.1956x; 1.0986x over previous
import functools
import jax
import jax.numpy as jnp
from jax import lax
from jax.experimental import pallas as pl
from jax.experimental.pallas import tpu as pltpu
from jax.experimental.pallas import tpu_sc as plsc

_POOL = 7
_MAX_TOK = 784.0
_MIN_TOK = 196.0

_HWS = ((100, 152), (50, 76), (25, 38), (13, 19))
_BASES = (0, 15200, 19000, 19952)
_TABLE_ROWS = 20224

_NB = 1024
_PER_WORKER = 32
_K = 16
_NPIX = _POOL * _POOL
_NJ = _NPIX * _K
_CHUNK = 112
_NCHUNK = _NJ // _CHUNK


def _pack_rows(t):
    u = lax.bitcast_convert_type(t.astype(jnp.bfloat16), jnp.uint16)
    u = u.astype(jnp.uint32)
    lo = jnp.concatenate([u[:, 0:16], u[:, 32:48]], axis=1)
    hi = jnp.concatenate([u[:, 16:32], u[:, 48:64]], axis=1)
    return lax.bitcast_convert_type(lo | (hi << 16), jnp.int32)


def _relayout_body(p0, p1, p2, p3, out):
    out[pl.ds(0, 15200), :] = _pack_rows(p0[...].T)
    out[pl.ds(15200, 3800), :] = _pack_rows(p1[...].T)
    out[pl.ds(19000, 952), :] = _pack_rows(p2[...].T)
    out[pl.ds(19952, 248), :] = _pack_rows(p3[...].T)


def _build_table(P0, P1, P2, P3):
    C = P0.shape[0]
    f0 = P0.reshape(C, -1)
    f1 = P1.reshape(C, -1)
    f2 = jnp.pad(P2.reshape(C, -1), ((0, 0), (0, 2)))
    f3 = jnp.pad(P3.reshape(C, -1), ((0, 0), (0, 1)))
    return pl.pallas_call(
        _relayout_body,
        out_shape=jax.ShapeDtypeStruct((_TABLE_ROWS, C // 2), jnp.int32),
    )(f0, f1, f2, f3)


def _corner(v, cidx, ext):
    v = jnp.maximum(v, 0.0)
    vl = jnp.floor(v)
    edge = vl >= (ext - 1.0)
    lo = jnp.where(edge, ext - 1.0, vl)
    hi = jnp.minimum(lo + 1.0, ext - 1.0)
    frac = jnp.where(edge, 0.0, v - vl)
    wgt = jnp.where(cidx == 1, frac, 1.0 - frac)
    pos = jnp.where(cidx == 1, hi, lo)
    return pos, wgt


def _addr_body(bb_ref, idx_ref, w_ref):
    bb = bb_ref[...]
    bx1 = bb[:, 0:1]
    by1 = bb[:, 1:2]
    bx2 = bb[:, 2:3]
    by2 = bb[:, 3:4]
    area = (bx2 - bx1) * (by2 - by1)

    choice = jnp.full_like(area, 3.0)
    for lvl in (2, 1, 0):
        h, w = _HWS[lvl]
        tok = area * float(h * w)
        m = (tok < _MAX_TOK) & (tok >= _MIN_TOK)
        choice = jnp.where(m, float(lvl), choice)

    def sel(vals):
        r = jnp.full_like(area, vals[3])
        for lvl in (2, 1, 0):
            r = jnp.where(choice == float(lvl), vals[lvl], r)
        return r

    Hc = sel([float(h) for h, _ in _HWS])
    Wc = sel([float(w) for _, w in _HWS])
    basec = sel([float(b) for b in _BASES])

    x1 = bx1 * Wc - 0.5
    y1 = by1 * Hc - 0.5
    bin_h = (by2 * Hc - 0.5 - y1) / float(_POOL)
    bin_w = (bx2 * Wc - 0.5 - x1) / float(_POOL)

    B = bb.shape[0]
    j = lax.broadcasted_iota(jnp.int32, (B, _NJ), 1)
    ph = (j // 112).astype(jnp.float32)
    pw = ((j // 16) % 7).astype(jnp.float32)
    sy = ((j // 8) % 2).astype(jnp.float32)
    cy = (j // 4) % 2
    sx = ((j // 2) % 2).astype(jnp.float32)
    cx = j % 2

    yf = y1 + (ph + (sy + 0.5) * 0.5) * bin_h
    xf = x1 + (pw + (sx + 0.5) * 0.5) * bin_w
    ypos, wy = _corner(yf, cy, Hc)
    xpos, wx = _corner(xf, cx, Wc)

    idx_ref[...] = (basec + ypos * Wc + xpos).astype(jnp.int32)
    w_ref[...] = wy * wx * 0.25


def _gen_addresses(bboxes_padded):
    grid = 8
    blk = _NB // grid
    return pl.pallas_call(
        _addr_body,
        grid=(grid,),
        in_specs=[pl.BlockSpec((blk, 4), lambda i: (i, 0))],
        out_specs=[
            pl.BlockSpec((blk, _NJ), lambda i: (i, 0)),
            pl.BlockSpec((blk, _NJ), lambda i: (i, 0)),
        ],
        out_shape=[
            jax.ShapeDtypeStruct((_NB, _NJ), jnp.int32),
            jax.ShapeDtypeStruct((_NB, _NJ), jnp.float32),
        ],
    )(bboxes_padded)


def _sc_body(table_hbm, idx_hbm, w_hbm, out_hbm, idx_v, w_v, rows_v, out_v,
             gsem0, gsem1, osem0, osem1):
    wid = lax.axis_index("s") * 2 + lax.axis_index("c")
    base = wid * _PER_WORKER
    gsems = (gsem0, gsem1)
    osems = (osem0, osem1)

    def fire(b, buf):
        pltpu.sync_copy(idx_hbm.at[b], idx_v.at[buf])
        pltpu.sync_copy(w_hbm.at[b], w_v.at[buf])
        for c in range(_NCHUNK):
            pltpu.async_copy(
                table_hbm.at[idx_v.at[buf, c]],
                rows_v.at[buf, pl.ds(c * _CHUNK, _CHUNK)],
                gsems[buf],
            )

    def drain_gather(buf):
        for c in range(_NCHUNK):
            pltpu.make_async_copy(
                table_hbm.at[idx_v.at[buf, c]],
                rows_v.at[buf, pl.ds(c * _CHUNK, _CHUNK)],
                gsems[buf],
            ).wait()

    def drain_out(b, buf):
        pltpu.make_async_copy(out_v.at[buf], out_hbm.at[b], osems[buf]).wait()

    def compute(b, buf):
        def pix_body(p, carry2):
            r0 = p * _K
            wvec = w_v[buf, pl.ds(r0, _K)]
            accs = [jnp.zeros((16,), jnp.float32) for _ in range(4)]
            for k in range(_K):
                ws = wvec[k]
                for h in range(2):
                    bc = rows_v[buf, r0 + k, pl.ds(h * 32, 32)]
                    va, vb = plsc.unpack(bc, format=plsc.PackFormat.INTERLEAVED)
                    accs[2 * h] = accs[2 * h] + ws * va
                    accs[2 * h + 1] = accs[2 * h + 1] + ws * vb
            for q in range(4):
                out_v[buf, p, pl.ds(q * 16, 16)] = accs[q]
            return carry2

        lax.fori_loop(0, _NPIX, pix_body, 0)
        pltpu.async_copy(out_v.at[buf], out_hbm.at[b], osems[buf])

    npair = _PER_WORKER // 2
    fire(base, 0)

    def pair_body(g, carry):
        b0 = base + 2 * g
        fire(b0 + 1, 1)
        drain_gather(0)

        @pl.when(g > 0)
        def _():
            drain_out(b0, 0)

        compute(b0, 0)

        @pl.when(g < npair - 1)
        def _():
            fire(b0 + 2, 0)

        drain_gather(1)

        @pl.when(g > 0)
        def _():
            drain_out(b0 + 1, 1)

        compute(b0 + 1, 1)
        return carry

    lax.fori_loop(0, npair, pair_body, 0)
    drain_out(base, 0)
    drain_out(base, 1)


def _sc_gather(table, idx, w):
    mesh = plsc.VectorSubcoreMesh(core_axis_name="c", subcore_axis_name="s")
    kern = functools.partial(
        pl.kernel,
        mesh=mesh,
        compiler_params=pltpu.CompilerParams(
            use_tc_tiling_on_sc=False, needs_layout_passes=False
        ),
        out_type=jax.ShapeDtypeStruct((_NB, _NPIX, 64), jnp.float32),
        scratch_types=[
            pltpu.VMEM((2, _NCHUNK, _CHUNK), jnp.int32),
            pltpu.VMEM((2, _NJ), jnp.float32),
            pltpu.VMEM((2, _NJ, 64), jnp.bfloat16),
            pltpu.VMEM((2, _NPIX, 64), jnp.float32),
            pltpu.SemaphoreType.DMA,
            pltpu.SemaphoreType.DMA,
            pltpu.SemaphoreType.DMA,
            pltpu.SemaphoreType.DMA,
        ],
    )(_sc_body)
    table_bf = lax.bitcast_convert_type(table, jnp.bfloat16).reshape(_TABLE_ROWS, 64)
    return kern(table_bf, idx.reshape(_NB, _NCHUNK, _CHUNK), w)


def _xpose_body(i_ref, o_ref):
    o_ref[...] = jnp.swapaxes(i_ref[...], 1, 2)


def _xpose(out_pc):
    grid = 16
    blk = _NB // grid
    return pl.pallas_call(
        _xpose_body,
        grid=(grid,),
        in_specs=[pl.BlockSpec((blk, _NPIX, 64), lambda i: (i, 0, 0))],
        out_specs=pl.BlockSpec((blk, 64, _NPIX), lambda i: (i, 0, 0)),
        out_shape=jax.ShapeDtypeStruct((_NB, 64, _NPIX), jnp.float32),
    )(out_pc)


def kernel(bboxes, P0, P1, P2, P3):
    N = bboxes.shape[0]
    table = _build_table(P0, P1, P2, P3)
    bb = jnp.pad(bboxes, ((0, _NB - N), (0, 0)))
    idx, w = _gen_addresses(bb)
    out = _sc_gather(table, idx, w)
    return _xpose(out).reshape(_NB, 64, _POOL, _POOL)[:N]


# speedup vs baseline: 92.6308x; 1.1002x over previous
import functools
import jax
import jax.numpy as jnp
from jax import lax
from jax.experimental import pallas as pl
from jax.experimental.pallas import tpu as pltpu
from jax.experimental.pallas import tpu_sc as plsc

_POOL = 7
_MAX_TOK = 784.0
_MIN_TOK = 196.0

_HWS = ((100, 152), (50, 76), (25, 38), (13, 19))
_BASES = (0, 15200, 19000, 19952)
_TABLE_ROWS = 20224

_NB = 1024
_PER_WORKER = 32
_K = 16
_NPIX = _POOL * _POOL
_NJ = _NPIX * _K
_CHUNK = 112
_NCHUNK = _NJ // _CHUNK


def _pack_rows(t):
    u = lax.bitcast_convert_type(t.astype(jnp.bfloat16), jnp.uint16)
    u = u.astype(jnp.uint32)
    lo = jnp.concatenate([u[:, 0:16], u[:, 32:48]], axis=1)
    hi = jnp.concatenate([u[:, 16:32], u[:, 48:64]], axis=1)
    return lax.bitcast_convert_type(lo | (hi << 16), jnp.int32)


def _relayout_body(p0, p1, p2, p3, out):
    out[pl.ds(0, 15200), :] = _pack_rows(p0[...].T)
    out[pl.ds(15200, 3800), :] = _pack_rows(p1[...].T)
    out[pl.ds(19000, 952), :] = _pack_rows(p2[...].T)
    out[pl.ds(19952, 248), :] = _pack_rows(p3[...].T)


def _build_table(P0, P1, P2, P3):
    C = P0.shape[0]
    f0 = P0.reshape(C, -1)
    f1 = P1.reshape(C, -1)
    f2 = jnp.pad(P2.reshape(C, -1), ((0, 0), (0, 2)))
    f3 = jnp.pad(P3.reshape(C, -1), ((0, 0), (0, 1)))
    return pl.pallas_call(
        _relayout_body,
        out_shape=jax.ShapeDtypeStruct((_TABLE_ROWS, C // 2), jnp.int32),
    )(f0, f1, f2, f3)


def _corner(v, cidx, ext):
    v = jnp.maximum(v, 0.0)
    vl = jnp.floor(v)
    edge = vl >= (ext - 1.0)
    lo = jnp.where(edge, ext - 1.0, vl)
    hi = jnp.minimum(lo + 1.0, ext - 1.0)
    frac = jnp.where(edge, 0.0, v - vl)
    wgt = jnp.where(cidx == 1, frac, 1.0 - frac)
    pos = jnp.where(cidx == 1, hi, lo)
    return pos, wgt


def _addr_body(bb_ref, idx_ref, w_ref):
    bb = bb_ref[...]
    bx1 = bb[:, 0:1]
    by1 = bb[:, 1:2]
    bx2 = bb[:, 2:3]
    by2 = bb[:, 3:4]
    area = (bx2 - bx1) * (by2 - by1)

    choice = jnp.full_like(area, 3.0)
    for lvl in (2, 1, 0):
        h, w = _HWS[lvl]
        tok = area * float(h * w)
        m = (tok < _MAX_TOK) & (tok >= _MIN_TOK)
        choice = jnp.where(m, float(lvl), choice)

    def sel(vals):
        r = jnp.full_like(area, vals[3])
        for lvl in (2, 1, 0):
            r = jnp.where(choice == float(lvl), vals[lvl], r)
        return r

    Hc = sel([float(h) for h, _ in _HWS])
    Wc = sel([float(w) for _, w in _HWS])
    basec = sel([float(b) for b in _BASES])

    x1 = bx1 * Wc - 0.5
    y1 = by1 * Hc - 0.5
    bin_h = (by2 * Hc - 0.5 - y1) / float(_POOL)
    bin_w = (bx2 * Wc - 0.5 - x1) / float(_POOL)

    B = bb.shape[0]
    j = lax.broadcasted_iota(jnp.int32, (B, _NJ), 1)
    ph = (j // 112).astype(jnp.float32)
    pw = ((j // 16) % 7).astype(jnp.float32)
    sy = ((j // 8) % 2).astype(jnp.float32)
    cy = (j // 4) % 2
    sx = ((j // 2) % 2).astype(jnp.float32)
    cx = j % 2

    yf = y1 + (ph + (sy + 0.5) * 0.5) * bin_h
    xf = x1 + (pw + (sx + 0.5) * 0.5) * bin_w
    ypos, wy = _corner(yf, cy, Hc)
    xpos, wx = _corner(xf, cx, Wc)

    idx_ref[...] = (basec + ypos * Wc + xpos).astype(jnp.int32)
    w_ref[...] = wy * wx * 0.25


def _gen_addresses(bboxes_padded):
    grid = 8
    blk = _NB // grid
    return pl.pallas_call(
        _addr_body,
        grid=(grid,),
        in_specs=[pl.BlockSpec((blk, 4), lambda i: (i, 0))],
        out_specs=[
            pl.BlockSpec((blk, _NJ), lambda i: (i, 0)),
            pl.BlockSpec((blk, _NJ), lambda i: (i, 0)),
        ],
        out_shape=[
            jax.ShapeDtypeStruct((_NB, _NJ), jnp.int32),
            jax.ShapeDtypeStruct((_NB, _NJ), jnp.float32),
        ],
    )(bboxes_padded)


def _sc_body(table_hbm, idx_hbm, w_hbm, out_hbm, idx_v, w_v, rows_v, out_v,
             gsem0, gsem1, osem0, osem1):
    wid = lax.axis_index("s") * 2 + lax.axis_index("c")
    base = wid * _PER_WORKER
    gsems = (gsem0, gsem1)
    osems = (osem0, osem1)

    def fire(b, buf):
        pltpu.sync_copy(idx_hbm.at[b], idx_v.at[buf])
        pltpu.sync_copy(w_hbm.at[b], w_v.at[buf])
        for c in range(_NCHUNK):
            pltpu.async_copy(
                table_hbm.at[idx_v.at[buf, c]],
                rows_v.at[buf, pl.ds(c * _CHUNK, _CHUNK)],
                gsems[buf],
            )

    def drain_gather(buf):
        for c in range(_NCHUNK):
            pltpu.make_async_copy(
                table_hbm.at[idx_v.at[buf, c]],
                rows_v.at[buf, pl.ds(c * _CHUNK, _CHUNK)],
                gsems[buf],
            ).wait()

    def drain_out(b, buf):
        pltpu.make_async_copy(out_v.at[buf], out_hbm.at[b], osems[buf]).wait()

    def compute(b, buf):
        def pix_body(p, carry2):
            r0 = p * _K
            wvec = w_v[buf, pl.ds(r0, _K)]
            accs = [jnp.zeros((16,), jnp.float32) for _ in range(4)]
            for k in range(_K):
                ws = wvec[k]
                for h in range(2):
                    bc = rows_v[buf, r0 + k, pl.ds(h * 32, 32)]
                    va, vb = plsc.unpack(bc, format=plsc.PackFormat.INTERLEAVED)
                    accs[2 * h] = accs[2 * h] + ws * va
                    accs[2 * h + 1] = accs[2 * h + 1] + ws * vb
            for q in range(4):
                out_v[buf, p, pl.ds(q * 16, 16)] = accs[q]
            return carry2

        lax.fori_loop(0, _NPIX, pix_body, 0)
        pltpu.async_copy(out_v.at[buf], out_hbm.at[b], osems[buf])

    npair = _PER_WORKER // 2
    fire(base, 0)

    def pair_body(g, carry):
        b0 = base + 2 * g
        fire(b0 + 1, 1)
        drain_gather(0)

        @pl.when(g > 0)
        def _():
            drain_out(b0, 0)

        compute(b0, 0)

        @pl.when(g < npair - 1)
        def _():
            fire(b0 + 2, 0)

        drain_gather(1)

        @pl.when(g > 0)
        def _():
            drain_out(b0 + 1, 1)

        compute(b0 + 1, 1)
        return carry

    lax.fori_loop(0, npair, pair_body, 0)
    drain_out(base, 0)
    drain_out(base, 1)


def _sc_gather(table, idx, w):
    mesh = plsc.VectorSubcoreMesh(core_axis_name="c", subcore_axis_name="s")
    kern = functools.partial(
        pl.kernel,
        mesh=mesh,
        compiler_params=pltpu.CompilerParams(
            use_tc_tiling_on_sc=False, needs_layout_passes=False
        ),
        out_type=jax.ShapeDtypeStruct((_NB, _NPIX, 64), jnp.float32),
        scratch_types=[
            pltpu.VMEM((2, _NCHUNK, _CHUNK), jnp.int32),
            pltpu.VMEM((2, _NJ), jnp.float32),
            pltpu.VMEM((2, _NJ, 64), jnp.bfloat16),
            pltpu.VMEM((2, _NPIX, 64), jnp.float32),
            pltpu.SemaphoreType.DMA,
            pltpu.SemaphoreType.DMA,
            pltpu.SemaphoreType.DMA,
            pltpu.SemaphoreType.DMA,
        ],
    )(_sc_body)
    table_bf = lax.bitcast_convert_type(table, jnp.bfloat16).reshape(_TABLE_ROWS, 64)
    return kern(table_bf, idx.reshape(_NB, _NCHUNK, _CHUNK), w)


def _xpose_body(i_ref, o_ref):
    ri = lax.broadcasted_iota(jnp.int32, (_NPIX, _NPIX), 0)
    ci = lax.broadcasted_iota(jnp.int32, (_NPIX, _NPIX), 1)
    eye = (ri == ci).astype(jnp.float32)
    o_ref[...] = lax.dot_general(
        i_ref[...], eye, (((1,), (0,)), ((), ())),
        preferred_element_type=jnp.float32,
    )


def _xpose(out_pc, n):
    grid = 16
    blk = _NB // grid
    return pl.pallas_call(
        _xpose_body,
        grid=(grid,),
        in_specs=[pl.BlockSpec((blk, _NPIX, 64), lambda i: (i, 0, 0))],
        out_specs=pl.BlockSpec((blk, 64, _NPIX), lambda i: (i, 0, 0)),
        out_shape=jax.ShapeDtypeStruct((n, 64, _NPIX), jnp.float32),
    )(out_pc)


def kernel(bboxes, P0, P1, P2, P3):
    N = bboxes.shape[0]
    table = _build_table(P0, P1, P2, P3)
    bb = jnp.pad(bboxes, ((0, _NB - N), (0, 0)))
    idx, w = _gen_addresses(bb)
    out = _sc_gather(table, idx, w)
    return _xpose(out, N).reshape(N, 64, _POOL, _POOL)


# speedup vs baseline: 100.4249x; 1.0841x over previous
import functools
import jax
import jax.numpy as jnp
from jax import lax
from jax.experimental import pallas as pl
from jax.experimental.pallas import tpu as pltpu
from jax.experimental.pallas import tpu_sc as plsc

_POOL = 7
_MAX_TOK = 784.0
_MIN_TOK = 196.0

_HWS = ((100, 152), (50, 76), (25, 38), (13, 19))
_BASES = (0, 15200, 19000, 19952)
_TABLE_ROWS = 20224

_NB = 1024
_PER_WORKER = 32
_K = 16
_NPIX = _POOL * _POOL
_NJ = _NPIX * _K
_CHUNK = 112
_NCHUNK = _NJ // _CHUNK


def _pack_rows(t):
    u = lax.bitcast_convert_type(t.astype(jnp.bfloat16), jnp.uint16)
    u = u.astype(jnp.uint32)
    lo = jnp.concatenate([u[:, 0:16], u[:, 32:48]], axis=1)
    hi = jnp.concatenate([u[:, 16:32], u[:, 48:64]], axis=1)
    return lax.bitcast_convert_type(lo | (hi << 16), jnp.int32)


def _relayout_body(p0, p1, p2, p3, out):
    out[pl.ds(0, 15200), :] = _pack_rows(p0[...].T)
    out[pl.ds(15200, 3800), :] = _pack_rows(p1[...].T)
    out[pl.ds(19000, 952), :] = _pack_rows(p2[...].T)
    out[pl.ds(19952, 248), :] = _pack_rows(p3[...].T)


def _build_table(P0, P1, P2, P3):
    C = P0.shape[0]
    f0 = P0.reshape(C, -1)
    f1 = P1.reshape(C, -1)
    f2 = jnp.pad(P2.reshape(C, -1), ((0, 0), (0, 2)))
    f3 = jnp.pad(P3.reshape(C, -1), ((0, 0), (0, 1)))
    return pl.pallas_call(
        _relayout_body,
        out_shape=jax.ShapeDtypeStruct((_TABLE_ROWS, C // 2), jnp.int32),
    )(f0, f1, f2, f3)


def _params_body(bb_ref, prm_ref):
    bb = bb_ref[...]
    bx1 = bb[:, 0:1]
    by1 = bb[:, 1:2]
    bx2 = bb[:, 2:3]
    by2 = bb[:, 3:4]
    area = (bx2 - bx1) * (by2 - by1)

    choice = jnp.full_like(area, 3.0)
    for lvl in (2, 1, 0):
        h, w = _HWS[lvl]
        tok = area * float(h * w)
        m = (tok < _MAX_TOK) & (tok >= _MIN_TOK)
        choice = jnp.where(m, float(lvl), choice)

    def sel(vals):
        r = jnp.full_like(area, vals[3])
        for lvl in (2, 1, 0):
            r = jnp.where(choice == float(lvl), vals[lvl], r)
        return r

    Hc = sel([float(h) for h, _ in _HWS])
    Wc = sel([float(w) for _, w in _HWS])
    basec = sel([float(b) for b in _BASES])

    x1 = bx1 * Wc - 0.5
    y1 = by1 * Hc - 0.5
    bin_h = (by2 * Hc - 0.5 - y1) / float(_POOL)
    bin_w = (bx2 * Wc - 0.5 - x1) / float(_POOL)

    prm = jnp.concatenate(
        [y1, x1, bin_h, bin_w, Hc - 1.0, Wc - 1.0, Wc, basec], axis=1
    )
    prm_ref[...] = jnp.broadcast_to(prm[:, :, None], prm.shape + (16,))


def _gen_params(bboxes_padded):
    grid = 8
    blk = _NB // grid
    return pl.pallas_call(
        _params_body,
        grid=(grid,),
        in_specs=[pl.BlockSpec((blk, 4), lambda i: (i, 0))],
        out_specs=pl.BlockSpec((blk, 8, 16), lambda i: (i, 0, 0)),
        out_shape=jax.ShapeDtypeStruct((_NB, 8, 16), jnp.float32),
    )(bboxes_padded)


def _sc_body(table_hbm, prm_hbm, out_hbm, prm_v, idx_v, w_v, rows_v, out_v,
             gsem0, gsem1, osem0, osem1):
    wid = lax.axis_index("s") * 2 + lax.axis_index("c")
    base = wid * _PER_WORKER
    gsems = (gsem0, gsem1)
    osems = (osem0, osem1)

    kl = lax.iota(jnp.int32, 16)
    syh = (((kl >> 3) & 1).astype(jnp.float32) * 0.5 + 0.25)
    sxh = (((kl >> 1) & 1).astype(jnp.float32) * 0.5 + 0.25)
    cym = ((kl >> 2) & 1) == 1
    cxm = (kl & 1) == 1

    def corner(v, cmask, em1f, em1i):
        v = jnp.maximum(v, 0.0)
        vli = v.astype(jnp.int32)
        vlf = vli.astype(jnp.float32)
        edge = vlf >= em1f
        lof = jnp.where(edge, em1f, vlf)
        loi = lof.astype(jnp.int32)
        hii = jnp.minimum(loi + 1, em1i)
        frac = jnp.where(edge, 0.0, v - vlf)
        wgt = jnp.where(cmask, frac, 1.0 - frac)
        pos = jnp.where(cmask, hii, loi)
        return pos, wgt

    def fire(b, buf):
        pltpu.sync_copy(prm_hbm.at[b], prm_v.at[buf])
        y1v = prm_v[buf, 0, :]
        x1v = prm_v[buf, 1, :]
        bhv = prm_v[buf, 2, :]
        bwv = prm_v[buf, 3, :]
        hm1f = prm_v[buf, 4, :]
        wm1f = prm_v[buf, 5, :]
        wfv = prm_v[buf, 6, :]
        basef = prm_v[buf, 7, :]
        hm1i = hm1f.astype(jnp.int32)
        wm1i = wm1f.astype(jnp.int32)
        wiv = wfv.astype(jnp.int32)
        basei = basef.astype(jnp.int32)

        def ph_body(ph, c1):
            phf = jnp.full((16,), ph, jnp.int32).astype(jnp.float32)
            yv = y1v + (phf + syh) * bhv
            ypos, wy = corner(yv, cym, hm1f, hm1i)
            yrow = basei + ypos * wiv

            def pw_body(pw, c2):
                pwf = jnp.full((16,), pw, jnp.int32).astype(jnp.float32)
                xv = x1v + (pwf + sxh) * bwv
                xpos, wx = corner(xv, cxm, wm1f, wm1i)
                idx_v[buf, ph, pl.ds(pw * 16, 16)] = yrow + xpos
                w_v[buf, pl.ds((ph * 7 + pw) * 16, 16)] = wy * wx * 0.25
                return c2

            lax.fori_loop(0, _POOL, pw_body, 0)
            return c1

        lax.fori_loop(0, _POOL, ph_body, 0)

        for c in range(_NCHUNK):
            pltpu.async_copy(
                table_hbm.at[idx_v.at[buf, c]],
                rows_v.at[buf, pl.ds(c * _CHUNK, _CHUNK)],
                gsems[buf],
            )

    def drain_gather(buf):
        for c in range(_NCHUNK):
            pltpu.make_async_copy(
                table_hbm.at[idx_v.at[buf, c]],
                rows_v.at[buf, pl.ds(c * _CHUNK, _CHUNK)],
                gsems[buf],
            ).wait()

    def drain_out(b, buf):
        pltpu.make_async_copy(out_v.at[buf], out_hbm.at[b], osems[buf]).wait()

    def compute(b, buf):
        def pix_body(p, carry2):
            r0 = p * _K
            wvec = w_v[buf, pl.ds(r0, _K)]
            accs = [jnp.zeros((16,), jnp.float32) for _ in range(4)]
            for k in range(_K):
                ws = wvec[k]
                for h in range(2):
                    bc = rows_v[buf, r0 + k, pl.ds(h * 32, 32)]
                    va, vb = plsc.unpack(bc, format=plsc.PackFormat.INTERLEAVED)
                    accs[2 * h] = accs[2 * h] + ws * va
                    accs[2 * h + 1] = accs[2 * h + 1] + ws * vb
            for q in range(4):
                out_v[buf, p, pl.ds(q * 16, 16)] = accs[q]
            return carry2

        lax.fori_loop(0, _NPIX, pix_body, 0)
        pltpu.async_copy(out_v.at[buf], out_hbm.at[b], osems[buf])

    npair = _PER_WORKER // 2
    fire(base, 0)

    def pair_body(g, carry):
        b0 = base + 2 * g
        fire(b0 + 1, 1)
        drain_gather(0)

        @pl.when(g > 0)
        def _():
            drain_out(b0, 0)

        compute(b0, 0)

        @pl.when(g < npair - 1)
        def _():
            fire(b0 + 2, 0)

        drain_gather(1)

        @pl.when(g > 0)
        def _():
            drain_out(b0 + 1, 1)

        compute(b0 + 1, 1)
        return carry

    lax.fori_loop(0, npair, pair_body, 0)
    drain_out(base, 0)
    drain_out(base, 1)


def _sc_gather(table, prm):
    mesh = plsc.VectorSubcoreMesh(core_axis_name="c", subcore_axis_name="s")
    kern = functools.partial(
        pl.kernel,
        mesh=mesh,
        compiler_params=pltpu.CompilerParams(
            use_tc_tiling_on_sc=False, needs_layout_passes=False
        ),
        out_type=jax.ShapeDtypeStruct((_NB, _NPIX, 64), jnp.float32),
        scratch_types=[
            pltpu.VMEM((2, 8, 16), jnp.float32),
            pltpu.VMEM((2, _NCHUNK, _CHUNK), jnp.int32),
            pltpu.VMEM((2, _NJ), jnp.float32),
            pltpu.VMEM((2, _NJ, 64), jnp.bfloat16),
            pltpu.VMEM((2, _NPIX, 64), jnp.float32),
            pltpu.SemaphoreType.DMA,
            pltpu.SemaphoreType.DMA,
            pltpu.SemaphoreType.DMA,
            pltpu.SemaphoreType.DMA,
        ],
    )(_sc_body)
    table_bf = lax.bitcast_convert_type(table, jnp.bfloat16).reshape(_TABLE_ROWS, 64)
    return kern(table_bf, prm)


def _xpose_body(i_ref, o_ref):
    ri = lax.broadcasted_iota(jnp.int32, (_NPIX, _NPIX), 0)
    ci = lax.broadcasted_iota(jnp.int32, (_NPIX, _NPIX), 1)
    eye = (ri == ci).astype(jnp.float32)
    o_ref[...] = lax.dot_general(
        i_ref[...], eye, (((1,), (0,)), ((), ())),
        preferred_element_type=jnp.float32,
    )


def _xpose(out_pc, n):
    grid = 16
    blk = _NB // grid
    return pl.pallas_call(
        _xpose_body,
        grid=(grid,),
        in_specs=[pl.BlockSpec((blk, _NPIX, 64), lambda i: (i, 0, 0))],
        out_specs=pl.BlockSpec((blk, 64, _NPIX), lambda i: (i, 0, 0)),
        out_shape=jax.ShapeDtypeStruct((n, 64, _NPIX), jnp.float32),
    )(out_pc)


def kernel(bboxes, P0, P1, P2, P3):
    N = bboxes.shape[0]
    table = _build_table(P0, P1, P2, P3)
    bb = jnp.pad(bboxes, ((0, _NB - N), (0, 0)))
    prm = _gen_params(bb)
    out = _sc_gather(table, prm)
    return _xpose(out, N).reshape(N, 64, _POOL, _POOL)


# speedup vs baseline: 101.0512x; 1.0062x over previous
import functools
import jax
import jax.numpy as jnp
from jax import lax
from jax.experimental import pallas as pl
from jax.experimental.pallas import tpu as pltpu
from jax.experimental.pallas import tpu_sc as plsc

_POOL = 7
_MAX_TOK = 784.0
_MIN_TOK = 196.0

_HWS = ((100, 152), (50, 76), (25, 38), (13, 19))
_BASES = (0, 15200, 19000, 19952)
_TABLE_ROWS = 20224

_NB = 1024
_NOUT = 1000
_PER_WORKER = 32
_K = 16
_NPIX = _POOL * _POOL
_NJ = _NPIX * _K
_CHUNK = 112
_NCHUNK = _NJ // _CHUNK


def _pack_rows(t):
    u = lax.bitcast_convert_type(t.astype(jnp.bfloat16), jnp.uint16)
    u = u.astype(jnp.uint32)
    lo = jnp.concatenate([u[:, 0:16], u[:, 32:48]], axis=1)
    hi = jnp.concatenate([u[:, 16:32], u[:, 48:64]], axis=1)
    return lax.bitcast_convert_type(lo | (hi << 16), jnp.int32)


def _relayout_body(p0, p1, p2, p3, out):
    out[pl.ds(0, 15200), :] = _pack_rows(p0[...].T)
    out[pl.ds(15200, 3800), :] = _pack_rows(p1[...].T)
    out[pl.ds(19000, 952), :] = _pack_rows(p2[...].T)
    out[pl.ds(19952, 248), :] = _pack_rows(p3[...].T)


def _build_table(P0, P1, P2, P3):
    C = P0.shape[0]
    f0 = P0.reshape(C, -1)
    f1 = P1.reshape(C, -1)
    f2 = jnp.pad(P2.reshape(C, -1), ((0, 0), (0, 2)))
    f3 = jnp.pad(P3.reshape(C, -1), ((0, 0), (0, 1)))
    return pl.pallas_call(
        _relayout_body,
        out_shape=jax.ShapeDtypeStruct((_TABLE_ROWS, C // 2), jnp.int32),
    )(f0, f1, f2, f3)


def _params_body(bb_ref, prm_ref):
    bb = bb_ref[...]
    bx1 = bb[:, 0:1]
    by1 = bb[:, 1:2]
    bx2 = bb[:, 2:3]
    by2 = bb[:, 3:4]
    area = (bx2 - bx1) * (by2 - by1)

    choice = jnp.full_like(area, 3.0)
    for lvl in (2, 1, 0):
        h, w = _HWS[lvl]
        tok = area * float(h * w)
        m = (tok < _MAX_TOK) & (tok >= _MIN_TOK)
        choice = jnp.where(m, float(lvl), choice)

    def sel(vals):
        r = jnp.full_like(area, vals[3])
        for lvl in (2, 1, 0):
            r = jnp.where(choice == float(lvl), vals[lvl], r)
        return r

    Hc = sel([float(h) for h, _ in _HWS])
    Wc = sel([float(w) for _, w in _HWS])
    basec = sel([float(b) for b in _BASES])

    x1 = bx1 * Wc - 0.5
    y1 = by1 * Hc - 0.5
    bin_h = (by2 * Hc - 0.5 - y1) / float(_POOL)
    bin_w = (bx2 * Wc - 0.5 - x1) / float(_POOL)

    prm = jnp.concatenate(
        [y1, x1, bin_h, bin_w, Hc - 1.0, Wc - 1.0, Wc, basec], axis=1
    )
    prm_ref[...] = jnp.broadcast_to(prm[:, :, None], prm.shape + (16,))


def _gen_params(bboxes_padded):
    grid = 8
    blk = _NB // grid
    return pl.pallas_call(
        _params_body,
        grid=(grid,),
        in_specs=[pl.BlockSpec((blk, 4), lambda i: (i, 0))],
        out_specs=pl.BlockSpec((blk, 8, 16), lambda i: (i, 0, 0)),
        out_shape=jax.ShapeDtypeStruct((_NB, 8, 16), jnp.float32),
    )(bboxes_padded)


def _sc_body(table_hbm, prm_hbm, out_hbm, prm_v, idx_v, w_v, rows_v, out_v,
             gsem0, gsem1, osem0, osem1):
    wid = lax.axis_index("s") * 2 + lax.axis_index("c")
    base = wid * _PER_WORKER
    gsems = (gsem0, gsem1)
    osems = (osem0, osem1)

    kl = lax.iota(jnp.int32, 16)
    syh = (((kl >> 3) & 1).astype(jnp.float32) * 0.5 + 0.25)
    sxh = (((kl >> 1) & 1).astype(jnp.float32) * 0.5 + 0.25)
    cym = ((kl >> 2) & 1) == 1
    cxm = (kl & 1) == 1

    def corner(v, cmask, em1f, em1i):
        v = jnp.maximum(v, 0.0)
        vli = v.astype(jnp.int32)
        vlf = vli.astype(jnp.float32)
        edge = vlf >= em1f
        lof = jnp.where(edge, em1f, vlf)
        loi = lof.astype(jnp.int32)
        hii = jnp.minimum(loi + 1, em1i)
        frac = jnp.where(edge, 0.0, v - vlf)
        wgt = jnp.where(cmask, frac, 1.0 - frac)
        pos = jnp.where(cmask, hii, loi)
        return pos, wgt

    def fire(b, buf):
        pltpu.sync_copy(prm_hbm.at[b], prm_v.at[buf])
        y1v = prm_v[buf, 0, :]
        x1v = prm_v[buf, 1, :]
        bhv = prm_v[buf, 2, :]
        bwv = prm_v[buf, 3, :]
        hm1f = prm_v[buf, 4, :]
        wm1f = prm_v[buf, 5, :]
        wfv = prm_v[buf, 6, :]
        basef = prm_v[buf, 7, :]
        hm1i = hm1f.astype(jnp.int32)
        wm1i = wm1f.astype(jnp.int32)
        wiv = wfv.astype(jnp.int32)
        basei = basef.astype(jnp.int32)

        def ph_body(ph, c1):
            phf = jnp.full((16,), ph, jnp.int32).astype(jnp.float32)
            yv = y1v + (phf + syh) * bhv
            ypos, wy = corner(yv, cym, hm1f, hm1i)
            yrow = basei + ypos * wiv

            def pw_body(pw, c2):
                pwf = jnp.full((16,), pw, jnp.int32).astype(jnp.float32)
                xv = x1v + (pwf + sxh) * bwv
                xpos, wx = corner(xv, cxm, wm1f, wm1i)
                idx_v[buf, ph, pl.ds(pw * 16, 16)] = yrow + xpos
                w_v[buf, pl.ds((ph * 7 + pw) * 16, 16)] = wy * wx * 0.25
                return c2

            lax.fori_loop(0, _POOL, pw_body, 0)
            return c1

        lax.fori_loop(0, _POOL, ph_body, 0)

        for c in range(_NCHUNK):
            pltpu.async_copy(
                table_hbm.at[idx_v.at[buf, c]],
                rows_v.at[buf, pl.ds(c * _CHUNK, _CHUNK)],
                gsems[buf],
            )

    def drain_gather(buf):
        for c in range(_NCHUNK):
            pltpu.make_async_copy(
                table_hbm.at[idx_v.at[buf, c]],
                rows_v.at[buf, pl.ds(c * _CHUNK, _CHUNK)],
                gsems[buf],
            ).wait()

    def drain_out(buf):
        pltpu.make_async_copy(out_v.at[buf], out_hbm.at[base], osems[buf]).wait()

    bufsplat = [jnp.full((16,), 0, jnp.int32), jnp.full((16,), 1, jnp.int32)]
    kl49 = kl * _NPIX

    def compute(b, buf):
        def pix_body(p, carry2):
            r0 = p * _K
            wvec = w_v[buf, pl.ds(r0, _K)]
            accs = [jnp.zeros((16,), jnp.float32) for _ in range(4)]
            for k in range(_K):
                ws = wvec[k]
                for h in range(2):
                    bc = rows_v[buf, r0 + k, pl.ds(h * 32, 32)]
                    va, vb = plsc.unpack(bc, format=plsc.PackFormat.INTERLEAVED)
                    accs[2 * h] = accs[2 * h] + ws * va
                    accs[2 * h + 1] = accs[2 * h + 1] + ws * vb
            for q in range(4):
                plsc.store_scatter(
                    out_v, [bufsplat[buf], kl49 + (q * 16 * _NPIX + p)], accs[q]
                )
            return carry2

        lax.fori_loop(0, _NPIX, pix_body, 0)

        @pl.when(b < _NOUT)
        def _():
            pltpu.async_copy(out_v.at[buf], out_hbm.at[b], osems[buf])

    npair = _PER_WORKER // 2
    fire(base, 0)

    def pair_body(g, carry):
        b0 = base + 2 * g
        fire(b0 + 1, 1)
        drain_gather(0)

        @pl.when((g > 0) & (b0 - 2 < _NOUT))
        def _():
            drain_out(0)

        compute(b0, 0)

        @pl.when(g < npair - 1)
        def _():
            fire(b0 + 2, 0)

        drain_gather(1)

        @pl.when((g > 0) & (b0 - 1 < _NOUT))
        def _():
            drain_out(1)

        compute(b0 + 1, 1)
        return carry

    lax.fori_loop(0, npair, pair_body, 0)
    if True:
        @pl.when(base + _PER_WORKER - 2 < _NOUT)
        def _():
            drain_out(0)

        @pl.when(base + _PER_WORKER - 1 < _NOUT)
        def _():
            drain_out(1)


def _sc_gather(table, prm):
    mesh = plsc.VectorSubcoreMesh(core_axis_name="c", subcore_axis_name="s")
    kern = functools.partial(
        pl.kernel,
        mesh=mesh,
        compiler_params=pltpu.CompilerParams(
            use_tc_tiling_on_sc=False, needs_layout_passes=False
        ),
        out_type=jax.ShapeDtypeStruct((_NOUT, 64 * _NPIX), jnp.float32),
        scratch_types=[
            pltpu.VMEM((2, 8, 16), jnp.float32),
            pltpu.VMEM((2, _NCHUNK, _CHUNK), jnp.int32),
            pltpu.VMEM((2, _NJ), jnp.float32),
            pltpu.VMEM((2, _NJ, 64), jnp.bfloat16),
            pltpu.VMEM((2, 64 * _NPIX), jnp.float32),
            pltpu.SemaphoreType.DMA,
            pltpu.SemaphoreType.DMA,
            pltpu.SemaphoreType.DMA,
            pltpu.SemaphoreType.DMA,
        ],
    )(_sc_body)
    table_bf = lax.bitcast_convert_type(table, jnp.bfloat16).reshape(_TABLE_ROWS, 64)
    return kern(table_bf, prm)


def kernel(bboxes, P0, P1, P2, P3):
    N = bboxes.shape[0]
    table = _build_table(P0, P1, P2, P3)
    bb = jnp.pad(bboxes, ((0, _NB - N), (0, 0)))
    prm = _gen_params(bb)
    out = _sc_gather(table, prm)
    return out.reshape(_NOUT, 64, _POOL, _POOL)[:N]


# speedup vs baseline: 106.2637x; 1.0516x over previous
import functools
import jax
import jax.numpy as jnp
from jax import lax
from jax.experimental import pallas as pl
from jax.experimental.pallas import tpu as pltpu
from jax.experimental.pallas import tpu_sc as plsc

_POOL = 7
_MAX_TOK = 784.0
_MIN_TOK = 196.0

_HWS = ((100, 152), (50, 76), (25, 38), (13, 19))
_BASES = (0, 15200, 19000, 19952)
_TABLE_ROWS = 20224

_NB = 1024
_NOUT = 1000
_PER_WORKER = 32
_K = 16
_NPIX = _POOL * _POOL
_NJ = _NPIX * _K
_CHUNK = 112
_NCHUNK = _NJ // _CHUNK


def _pack_rows(t):
    u = lax.bitcast_convert_type(t.astype(jnp.bfloat16), jnp.uint16)
    u = u.astype(jnp.uint32)
    lo = jnp.concatenate([u[:, 0:16], u[:, 32:48]], axis=1)
    hi = jnp.concatenate([u[:, 16:32], u[:, 48:64]], axis=1)
    return lax.bitcast_convert_type(lo | (hi << 16), jnp.int32)


def _relayout_body(p0, p1, p2, p3, out):
    out[pl.ds(0, 15200), :] = _pack_rows(p0[...].T)
    out[pl.ds(15200, 3800), :] = _pack_rows(p1[...].T)
    out[pl.ds(19000, 952), :] = _pack_rows(p2[...].T)
    out[pl.ds(19952, 248), :] = _pack_rows(p3[...].T)


def _build_table(P0, P1, P2, P3):
    C = P0.shape[0]
    f0 = P0.reshape(C, -1)
    f1 = P1.reshape(C, -1)
    f2 = jnp.pad(P2.reshape(C, -1), ((0, 0), (0, 2)))
    f3 = jnp.pad(P3.reshape(C, -1), ((0, 0), (0, 1)))
    return pl.pallas_call(
        _relayout_body,
        out_shape=jax.ShapeDtypeStruct((_TABLE_ROWS, C // 2), jnp.int32),
    )(f0, f1, f2, f3)


def _params_body(bb_ref, prm_ref):
    bb = bb_ref[...]
    bx1 = bb[:, 0:1]
    by1 = bb[:, 1:2]
    bx2 = bb[:, 2:3]
    by2 = bb[:, 3:4]
    area = (bx2 - bx1) * (by2 - by1)

    choice = jnp.full_like(area, 3.0)
    for lvl in (2, 1, 0):
        h, w = _HWS[lvl]
        tok = area * float(h * w)
        m = (tok < _MAX_TOK) & (tok >= _MIN_TOK)
        choice = jnp.where(m, float(lvl), choice)

    def sel(vals):
        r = jnp.full_like(area, vals[3])
        for lvl in (2, 1, 0):
            r = jnp.where(choice == float(lvl), vals[lvl], r)
        return r

    Hc = sel([float(h) for h, _ in _HWS])
    Wc = sel([float(w) for _, w in _HWS])
    basec = sel([float(b) for b in _BASES])

    x1 = bx1 * Wc - 0.5
    y1 = by1 * Hc - 0.5
    bin_h = (by2 * Hc - 0.5 - y1) / float(_POOL)
    bin_w = (bx2 * Wc - 0.5 - x1) / float(_POOL)

    prm = jnp.concatenate(
        [y1, x1, bin_h, bin_w, Hc - 1.0, Wc - 1.0, Wc, basec], axis=1
    )
    prm_ref[...] = jnp.broadcast_to(prm[:, :, None], prm.shape + (16,))


def _gen_params(bboxes_padded):
    grid = 8
    blk = _NB // grid
    return pl.pallas_call(
        _params_body,
        grid=(grid,),
        in_specs=[pl.BlockSpec((blk, 4), lambda i: (i, 0))],
        out_specs=pl.BlockSpec((blk, 8, 16), lambda i: (i, 0, 0)),
        out_shape=jax.ShapeDtypeStruct((_NB, 8, 16), jnp.float32),
    )(bboxes_padded)


def _sc_body(table_hbm, prm_hbm, out_hbm, prm_v, idx_v, w_v, rows_v, out_v,
             gsem0, gsem1, osem0, osem1):
    wid = lax.axis_index("s") * 2 + lax.axis_index("c")
    base = wid * _PER_WORKER
    gsems = (gsem0, gsem1)
    osems = (osem0, osem1)

    kl = lax.iota(jnp.int32, 16)
    syh = (((kl >> 3) & 1).astype(jnp.float32) * 0.5 + 0.25)
    sxh = (((kl >> 1) & 1).astype(jnp.float32) * 0.5 + 0.25)
    cym = ((kl >> 2) & 1) == 1
    cxm = (kl & 1) == 1

    def corner(v, cmask, em1f, em1i):
        v = jnp.maximum(v, 0.0)
        vli = v.astype(jnp.int32)
        vlf = vli.astype(jnp.float32)
        edge = vlf >= em1f
        lof = jnp.where(edge, em1f, vlf)
        loi = lof.astype(jnp.int32)
        hii = jnp.minimum(loi + 1, em1i)
        frac = jnp.where(edge, 0.0, v - vlf)
        wgt = jnp.where(cmask, frac, 1.0 - frac)
        pos = jnp.where(cmask, hii, loi)
        return pos, wgt

    def fire(b, buf):
        pltpu.sync_copy(prm_hbm.at[b], prm_v.at[buf])
        y1v = prm_v[buf, 0, :]
        x1v = prm_v[buf, 1, :]
        bhv = prm_v[buf, 2, :]
        bwv = prm_v[buf, 3, :]
        hm1f = prm_v[buf, 4, :]
        wm1f = prm_v[buf, 5, :]
        wfv = prm_v[buf, 6, :]
        basef = prm_v[buf, 7, :]
        hm1i = hm1f.astype(jnp.int32)
        wm1i = wm1f.astype(jnp.int32)
        wiv = wfv.astype(jnp.int32)
        basei = basef.astype(jnp.int32)

        def ph_body(ph, c1):
            phf = jnp.full((16,), ph, jnp.int32).astype(jnp.float32)
            yv = y1v + (phf + syh) * bhv
            ypos, wy = corner(yv, cym, hm1f, hm1i)
            yrow = basei + ypos * wiv

            def pw_body(pw, c2):
                pwf = jnp.full((16,), pw, jnp.int32).astype(jnp.float32)
                xv = x1v + (pwf + sxh) * bwv
                xpos, wx = corner(xv, cxm, wm1f, wm1i)
                idx_v[buf, ph, pl.ds(pw * 16, 16)] = yrow + xpos
                w_v[buf, pl.ds((ph * 7 + pw) * 16, 16)] = wy * wx * 0.25
                return c2

            lax.fori_loop(0, _POOL, pw_body, 0)
            return c1

        lax.fori_loop(0, _POOL, ph_body, 0)

        for c in range(_NCHUNK):
            pltpu.async_copy(
                table_hbm.at[idx_v.at[buf, c]],
                rows_v.at[buf, pl.ds(c * _CHUNK, _CHUNK)],
                gsems[buf],
            )

    def drain_gather(buf):
        for c in range(_NCHUNK):
            pltpu.make_async_copy(
                table_hbm.at[idx_v.at[buf, c]],
                rows_v.at[buf, pl.ds(c * _CHUNK, _CHUNK)],
                gsems[buf],
            ).wait()

    def drain_out(buf):
        pltpu.make_async_copy(out_v.at[buf], out_hbm.at[base], osems[buf]).wait()

    bufsplat = [jnp.full((16,), 0, jnp.int32), jnp.full((16,), 1, jnp.int32)]
    kl49 = kl * _NPIX

    def compute(b, buf):
        def pix_body(p, carry2):
            r0 = p * _K
            wvec = w_v[buf, pl.ds(r0, _K)]
            accs = [jnp.zeros((16,), jnp.float32) for _ in range(4)]
            for k in range(_K):
                ws = wvec[k]
                for h in range(2):
                    bc = rows_v[buf, r0 + k, pl.ds(h * 32, 32)]
                    va, vb = plsc.unpack(bc, format=plsc.PackFormat.INTERLEAVED)
                    accs[2 * h] = accs[2 * h] + ws * va
                    accs[2 * h + 1] = accs[2 * h + 1] + ws * vb
            for q in range(4):
                plsc.store_scatter(
                    out_v, [bufsplat[buf], kl49 + (q * 16 * _NPIX + p)], accs[q]
                )
            return carry2

        lax.fori_loop(0, _NPIX, pix_body, 0)

        @pl.when(b < _NOUT)
        def _():
            pltpu.async_copy(out_v.at[buf], out_hbm.at[b], osems[buf])

    npair = _PER_WORKER // 2
    fire(base, 0)

    def pair_body(g, carry):
        b0 = base + 2 * g
        fire(b0 + 1, 1)
        drain_gather(0)

        @pl.when((g > 0) & (b0 - 2 < _NOUT))
        def _():
            drain_out(0)

        compute(b0, 0)

        @pl.when(g < npair - 1)
        def _():
            fire(b0 + 2, 0)

        drain_gather(1)

        @pl.when((g > 0) & (b0 - 1 < _NOUT))
        def _():
            drain_out(1)

        compute(b0 + 1, 1)
        return carry

    lax.fori_loop(0, npair, pair_body, 0)
    if True:
        @pl.when(base + _PER_WORKER - 2 < _NOUT)
        def _():
            drain_out(0)

        @pl.when(base + _PER_WORKER - 1 < _NOUT)
        def _():
            drain_out(1)


def _sc_gather(table, prm):
    mesh = plsc.VectorSubcoreMesh(core_axis_name="c", subcore_axis_name="s")
    kern = functools.partial(
        pl.kernel,
        mesh=mesh,
        compiler_params=pltpu.CompilerParams(
            use_tc_tiling_on_sc=False, needs_layout_passes=False
        ),
        out_type=jax.ShapeDtypeStruct((_NOUT, 64 * _NPIX), jnp.float32),
        scratch_types=[
            pltpu.VMEM((2, 8, 16), jnp.float32),
            pltpu.VMEM((2, _NCHUNK, _CHUNK), jnp.int32),
            pltpu.VMEM((2, _NJ), jnp.float32),
            pltpu.VMEM((2, _NJ, 64), jnp.bfloat16),
            pltpu.VMEM((2, 64 * _NPIX), jnp.float32),
            pltpu.SemaphoreType.DMA,
            pltpu.SemaphoreType.DMA,
            pltpu.SemaphoreType.DMA,
            pltpu.SemaphoreType.DMA,
        ],
    )(_sc_body)
    table_bf = lax.bitcast_convert_type(table, jnp.bfloat16).reshape(_TABLE_ROWS, 64)
    return kern(table_bf, prm)


def _tc_copy_body(i_ref, o_ref):
    o_ref[...] = i_ref[...]


def _tc_copy(x):
    grid = 5
    blk = _NOUT // grid
    return pl.pallas_call(
        _tc_copy_body,
        grid=(grid,),
        in_specs=[pl.BlockSpec((blk, 64 * _NPIX), lambda i: (i, 0))],
        out_specs=pl.BlockSpec((blk, 64 * _NPIX), lambda i: (i, 0)),
        out_shape=jax.ShapeDtypeStruct((_NOUT, 64 * _NPIX), jnp.float32),
    )(x)


def kernel(bboxes, P0, P1, P2, P3):
    N = bboxes.shape[0]
    table = _build_table(P0, P1, P2, P3)
    bb = jnp.pad(bboxes, ((0, _NB - N), (0, 0)))
    prm = _gen_params(bb)
    out = _tc_copy(_sc_gather(table, prm))
    return out.reshape(_NOUT, 64, _POOL, _POOL)[:N]


# speedup vs baseline: 110.8264x; 1.0429x over previous
import functools
import jax
import jax.numpy as jnp
from jax import lax
from jax.experimental import pallas as pl
from jax.experimental.pallas import tpu as pltpu
from jax.experimental.pallas import tpu_sc as plsc

_POOL = 7
_MAX_TOK = 784.0
_MIN_TOK = 196.0

_HWS = ((100, 152), (50, 76), (25, 38), (13, 19))
_BASES = (0, 15200, 19000, 19952)
_TABLE_ROWS = 20224

_NB = 1024
_NOUT = 1000
_PER_WORKER = 32
_K = 16
_NPIX = _POOL * _POOL
_NJ = _NPIX * _K
_CHUNK = 112
_NCHUNK = _NJ // _CHUNK


def _pack_rows(t):
    u = lax.bitcast_convert_type(t.astype(jnp.bfloat16), jnp.uint16)
    u = u.astype(jnp.uint32)
    lo = jnp.concatenate([u[:, 0:16], u[:, 32:48]], axis=1)
    hi = jnp.concatenate([u[:, 16:32], u[:, 48:64]], axis=1)
    return lax.bitcast_convert_type(lo | (hi << 16), jnp.int32)


def _relayout_body(p0, p1, p2, p3, out):
    out[pl.ds(0, 15200), :] = _pack_rows(p0[...].T)
    out[pl.ds(15200, 3800), :] = _pack_rows(p1[...].T)
    out[pl.ds(19000, 952), :] = _pack_rows(p2[...].T)
    out[pl.ds(19952, 248), :] = _pack_rows(p3[...].T)


def _build_table(P0, P1, P2, P3):
    C = P0.shape[0]
    f0 = P0.reshape(C, -1)
    f1 = P1.reshape(C, -1)
    f2 = jnp.pad(P2.reshape(C, -1), ((0, 0), (0, 2)))
    f3 = jnp.pad(P3.reshape(C, -1), ((0, 0), (0, 1)))
    return pl.pallas_call(
        _relayout_body,
        out_shape=jax.ShapeDtypeStruct((_TABLE_ROWS, C // 2), jnp.int32),
    )(f0, f1, f2, f3)


def _params_body(bb_ref, prm_ref):
    bb = bb_ref[...]
    bx1 = bb[:, 0:1]
    by1 = bb[:, 1:2]
    bx2 = bb[:, 2:3]
    by2 = bb[:, 3:4]
    area = (bx2 - bx1) * (by2 - by1)

    choice = jnp.full_like(area, 3.0)
    for lvl in (2, 1, 0):
        h, w = _HWS[lvl]
        tok = area * float(h * w)
        m = (tok < _MAX_TOK) & (tok >= _MIN_TOK)
        choice = jnp.where(m, float(lvl), choice)

    def sel(vals):
        r = jnp.full_like(area, vals[3])
        for lvl in (2, 1, 0):
            r = jnp.where(choice == float(lvl), vals[lvl], r)
        return r

    Hc = sel([float(h) for h, _ in _HWS])
    Wc = sel([float(w) for _, w in _HWS])
    basec = sel([float(b) for b in _BASES])

    x1 = bx1 * Wc - 0.5
    y1 = by1 * Hc - 0.5
    bin_h = (by2 * Hc - 0.5 - y1) / float(_POOL)
    bin_w = (bx2 * Wc - 0.5 - x1) / float(_POOL)

    prm = jnp.concatenate(
        [y1, x1, bin_h, bin_w, Hc - 1.0, Wc - 1.0, Wc, basec], axis=1
    )
    prm_ref[...] = jnp.broadcast_to(prm[:, :, None], prm.shape + (16,))


def _gen_params(bboxes_padded):
    grid = 8
    blk = _NB // grid
    return pl.pallas_call(
        _params_body,
        grid=(grid,),
        in_specs=[pl.BlockSpec((blk, 4), lambda i: (i, 0))],
        out_specs=pl.BlockSpec((blk, 8, 16), lambda i: (i, 0, 0)),
        out_shape=jax.ShapeDtypeStruct((_NB, 8, 16), jnp.float32),
    )(bboxes_padded)


def _sc_body(table_hbm, prm_hbm, out_hbm, prm_v, idx_v, w_v, rows_v, out_v,
             gsem0, gsem1, osem0, osem1):
    wid = lax.axis_index("s") * 2 + lax.axis_index("c")
    base = wid * _PER_WORKER
    gsems = (gsem0, gsem1)
    osems = (osem0, osem1)

    kl = lax.iota(jnp.int32, 16)
    syh = (((kl >> 3) & 1).astype(jnp.float32) * 0.5 + 0.25)
    sxh = (((kl >> 1) & 1).astype(jnp.float32) * 0.5 + 0.25)
    cym = ((kl >> 2) & 1) == 1
    cxm = (kl & 1) == 1

    def corner(v, cmask, em1f, em1i):
        v = jnp.maximum(v, 0.0)
        vli = v.astype(jnp.int32)
        vlf = vli.astype(jnp.float32)
        edge = vlf >= em1f
        lof = jnp.where(edge, em1f, vlf)
        loi = lof.astype(jnp.int32)
        hii = jnp.minimum(loi + 1, em1i)
        frac = jnp.where(edge, 0.0, v - vlf)
        wgt = jnp.where(cmask, frac, 1.0 - frac)
        pos = jnp.where(cmask, hii, loi)
        return pos, wgt

    def fire(b, buf):
        pltpu.sync_copy(prm_hbm.at[b], prm_v.at[buf])
        y1v = prm_v[buf, 0, :]
        x1v = prm_v[buf, 1, :]
        bhv = prm_v[buf, 2, :]
        bwv = prm_v[buf, 3, :]
        hm1f = prm_v[buf, 4, :]
        wm1f = prm_v[buf, 5, :]
        wfv = prm_v[buf, 6, :]
        basef = prm_v[buf, 7, :]
        hm1i = hm1f.astype(jnp.int32)
        wm1i = wm1f.astype(jnp.int32)
        wiv = wfv.astype(jnp.int32)
        basei = basef.astype(jnp.int32)

        def ph_body(ph, c1):
            phf = jnp.full((16,), ph, jnp.int32).astype(jnp.float32)
            yv = y1v + (phf + syh) * bhv
            ypos, wy = corner(yv, cym, hm1f, hm1i)
            yrow = basei + ypos * wiv

            def pw_body(pw, c2):
                pwf = jnp.full((16,), pw, jnp.int32).astype(jnp.float32)
                xv = x1v + (pwf + sxh) * bwv
                xpos, wx = corner(xv, cxm, wm1f, wm1i)
                idx_v[buf, ph, pl.ds(pw * 16, 16)] = yrow + xpos
                w_v[buf, pl.ds((ph * 7 + pw) * 16, 16)] = wy * wx * 0.25
                return c2

            lax.fori_loop(0, _POOL, pw_body, 0)
            return c1

        lax.fori_loop(0, _POOL, ph_body, 0)

        for c in range(_NCHUNK):
            pltpu.async_copy(
                table_hbm.at[idx_v.at[buf, c]],
                rows_v.at[buf, pl.ds(c * _CHUNK, _CHUNK)],
                gsems[buf],
            )

    def drain_gather(buf):
        for c in range(_NCHUNK):
            pltpu.make_async_copy(
                table_hbm.at[idx_v.at[buf, c]],
                rows_v.at[buf, pl.ds(c * _CHUNK, _CHUNK)],
                gsems[buf],
            ).wait()

    def drain_out(buf):
        pltpu.make_async_copy(out_v.at[buf], out_hbm.at[base], osems[buf]).wait()

    bufsplat = [jnp.full((16,), 0, jnp.int32), jnp.full((16,), 1, jnp.int32)]
    kl49 = kl * _NPIX

    def compute(b, buf):
        def pix_body(p, carry2):
            r0 = p * _K
            wvec = w_v[buf, pl.ds(r0, _K)]
            acca = [jnp.zeros((16,), jnp.float32) for _ in range(4)]
            accb = [jnp.zeros((16,), jnp.float32) for _ in range(4)]
            for k in range(_K):
                ws = wvec[k]
                accs = acca if (k & 1) == 0 else accb
                for h in range(2):
                    bc = rows_v[buf, r0 + k, pl.ds(h * 32, 32)]
                    va, vb = plsc.unpack(bc, format=plsc.PackFormat.INTERLEAVED)
                    accs[2 * h] = accs[2 * h] + ws * va
                    accs[2 * h + 1] = accs[2 * h + 1] + ws * vb
            for q in range(4):
                plsc.store_scatter(
                    out_v,
                    [bufsplat[buf], kl49 + (q * 16 * _NPIX + p)],
                    acca[q] + accb[q],
                )
            return carry2

        lax.fori_loop(0, _NPIX, pix_body, 0)

        @pl.when(b < _NOUT)
        def _():
            pltpu.async_copy(out_v.at[buf], out_hbm.at[b], osems[buf])

    npair = _PER_WORKER // 2
    fire(base, 0)

    def pair_body(g, carry):
        b0 = base + 2 * g
        fire(b0 + 1, 1)
        drain_gather(0)

        @pl.when((g > 0) & (b0 - 2 < _NOUT))
        def _():
            drain_out(0)

        compute(b0, 0)

        @pl.when(g < npair - 1)
        def _():
            fire(b0 + 2, 0)

        drain_gather(1)

        @pl.when((g > 0) & (b0 - 1 < _NOUT))
        def _():
            drain_out(1)

        compute(b0 + 1, 1)
        return carry

    lax.fori_loop(0, npair, pair_body, 0)
    if True:
        @pl.when(base + _PER_WORKER - 2 < _NOUT)
        def _():
            drain_out(0)

        @pl.when(base + _PER_WORKER - 1 < _NOUT)
        def _():
            drain_out(1)


def _sc_gather(table, prm):
    mesh = plsc.VectorSubcoreMesh(core_axis_name="c", subcore_axis_name="s")
    kern = functools.partial(
        pl.kernel,
        mesh=mesh,
        compiler_params=pltpu.CompilerParams(
            use_tc_tiling_on_sc=False, needs_layout_passes=False
        ),
        out_type=jax.ShapeDtypeStruct((_NOUT, 64 * _NPIX), jnp.float32),
        scratch_types=[
            pltpu.VMEM((2, 8, 16), jnp.float32),
            pltpu.VMEM((2, _NCHUNK, _CHUNK), jnp.int32),
            pltpu.VMEM((2, _NJ), jnp.float32),
            pltpu.VMEM((2, _NJ, 64), jnp.bfloat16),
            pltpu.VMEM((2, 64 * _NPIX), jnp.float32),
            pltpu.SemaphoreType.DMA,
            pltpu.SemaphoreType.DMA,
            pltpu.SemaphoreType.DMA,
            pltpu.SemaphoreType.DMA,
        ],
    )(_sc_body)
    table_bf = lax.bitcast_convert_type(table, jnp.bfloat16).reshape(_TABLE_ROWS, 64)
    return kern(table_bf, prm)


def _tc_copy_body(i_ref, o_ref):
    o_ref[...] = i_ref[...]


def _tc_copy(x):
    grid = 5
    blk = _NOUT // grid
    return pl.pallas_call(
        _tc_copy_body,
        grid=(grid,),
        in_specs=[pl.BlockSpec((blk, 64 * _NPIX), lambda i: (i, 0))],
        out_specs=pl.BlockSpec((blk, 64 * _NPIX), lambda i: (i, 0)),
        out_shape=jax.ShapeDtypeStruct((_NOUT, 64 * _NPIX), jnp.float32),
    )(x)


def kernel(bboxes, P0, P1, P2, P3):
    N = bboxes.shape[0]
    table = _build_table(P0, P1, P2, P3)
    bb = jnp.pad(bboxes, ((0, _NB - N), (0, 0)))
    prm = _gen_params(bb)
    out = _tc_copy(_sc_gather(table, prm))
    return out.reshape(_NOUT, 64, _POOL, _POOL)[:N]
